# baseline (speedup 1.0000x reference)
import jax
import jax.numpy as jnp
from jax.experimental import pallas as pl
from jax.experimental.pallas import tpu as pltpu

_TOKENS = 8192
_DIM = 2048
_EXPERTS = 64
_BT = 1024
_BK = 1024
_NK = _DIM // _BK


def _gate_kernel(x_ref, w_ref, out_ref, acc_ref):
    k = pl.program_id(1)
    part = jax.lax.dot_general(
        x_ref[...], w_ref[...],
        dimension_numbers=(((1,), (1,)), ((), ())),
        preferred_element_type=jnp.float32)

    @pl.when(k == 0)
    def _():
        acc_ref[...] = part

    @pl.when(k > 0)
    def _():
        acc_ref[...] += part

    @pl.when(k == _NK - 1)
    def _():
        logits = acc_ref[...]
        m = jnp.max(logits, axis=1, keepdims=True)
        e = jnp.exp(logits - m)
        out_ref[...] = e / jnp.sum(e, axis=1, keepdims=True)


def kernel(inputs, wg):
    return pl.pallas_call(
        _gate_kernel,
        grid=(_TOKENS // _BT, _NK),
        in_specs=[
            pl.BlockSpec((_BT, _BK), lambda i, k: (i, k)),
            pl.BlockSpec((_EXPERTS, _BK), lambda i, k: (0, k)),
        ],
        out_specs=pl.BlockSpec((_BT, _EXPERTS), lambda i, k: (i, 0)),
        out_shape=jax.ShapeDtypeStruct((_TOKENS, _EXPERTS), jnp.float32),
        scratch_shapes=[pltpu.VMEM((_BT, _EXPERTS), jnp.float32)],
        compiler_params=pltpu.CompilerParams(
            dimension_semantics=("parallel", "arbitrary")),
    )(inputs, wg)


# speedup vs baseline: 1.1274x; 1.1274x over previous
import jax
import jax.numpy as jnp
from jax.experimental import pallas as pl
from jax.experimental.pallas import tpu as pltpu

_TOKENS = 8192
_DIM = 2048
_EXPERTS = 64
_BT = 512
_NCH = _TOKENS // _BT
_NBUF = 4


def _gate_kernel(x_hbm, w_ref, out_hbm, xbuf, obuf, in_sems, out_sems):
    w = w_ref[...]
    dn = (((1,), (1,)), ((), ()))

    def in_copy(c):
        slot = c % _NBUF
        return pltpu.make_async_copy(
            x_hbm.at[pl.ds(c * _BT, _BT), :], xbuf.at[slot], in_sems.at[slot])

    def out_copy(c):
        slot = c % _NBUF
        return pltpu.make_async_copy(
            obuf.at[slot], out_hbm.at[pl.ds(c * _BT, _BT), :],
            out_sems.at[slot])

    for c in range(_NBUF):
        in_copy(c).start()

    for c in range(_NCH):
        slot = c % _NBUF
        in_copy(c).wait()
        logits = jax.lax.dot_general(xbuf[slot], w, dimension_numbers=dn,
                                     preferred_element_type=jnp.float32)
        m = jnp.max(logits, axis=1, keepdims=True)
        e = jnp.exp(logits - m)
        if c >= _NBUF:
            out_copy(c - _NBUF).wait()
        obuf[slot] = e / jnp.sum(e, axis=1, keepdims=True)
        out_copy(c).start()
        nxt = c + _NBUF
        if nxt < _NCH:
            in_copy(nxt).start()

    for c in range(_NCH - _NBUF, _NCH):
        out_copy(c).wait()


def kernel(inputs, wg):
    return pl.pallas_call(
        _gate_kernel,
        in_specs=[
            pl.BlockSpec(memory_space=pltpu.MemorySpace.HBM),
            pl.BlockSpec((_EXPERTS, _DIM), lambda: (0, 0)),
        ],
        out_specs=pl.BlockSpec(memory_space=pltpu.MemorySpace.HBM),
        out_shape=jax.ShapeDtypeStruct((_TOKENS, _EXPERTS), jnp.float32),
        scratch_shapes=[
            pltpu.VMEM((_NBUF, _BT, _DIM), jnp.float32),
            pltpu.VMEM((_NBUF, _BT, _EXPERTS), jnp.float32),
            pltpu.SemaphoreType.DMA((_NBUF,)),
            pltpu.SemaphoreType.DMA((_NBUF,)),
        ],
    )(inputs, wg)


# speedup vs baseline: 1.1345x; 1.0062x over previous
import jax
import jax.numpy as jnp
from jax.experimental import pallas as pl
from jax.experimental.pallas import tpu as pltpu

_TOKENS = 8192
_DIM = 2048
_EXPERTS = 64
_BT = 1024
_NCH = _TOKENS // _BT
_NBUF = 4


def _gate_kernel(x_hbm, w_ref, out_hbm, xbuf, obuf, in_sems, out_sems):
    w = w_ref[...]
    dn = (((1,), (1,)), ((), ()))

    def in_copy(c):
        slot = c % _NBUF
        return pltpu.make_async_copy(
            x_hbm.at[pl.ds(c * _BT, _BT), :], xbuf.at[slot], in_sems.at[slot])

    def out_copy(c):
        slot = c % _NBUF
        return pltpu.make_async_copy(
            obuf.at[slot], out_hbm.at[pl.ds(c * _BT, _BT), :],
            out_sems.at[slot])

    for c in range(_NBUF):
        in_copy(c).start()

    for c in range(_NCH):
        slot = c % _NBUF
        in_copy(c).wait()
        logits = jax.lax.dot_general(xbuf[slot], w, dimension_numbers=dn,
                                     preferred_element_type=jnp.float32)
        m = jnp.max(logits, axis=1, keepdims=True)
        e = jnp.exp(logits - m)
        if c >= _NBUF:
            out_copy(c - _NBUF).wait()
        obuf[slot] = e / jnp.sum(e, axis=1, keepdims=True)
        out_copy(c).start()
        nxt = c + _NBUF
        if nxt < _NCH:
            in_copy(nxt).start()

    for c in range(_NCH - _NBUF, _NCH):
        out_copy(c).wait()


def kernel(inputs, wg):
    return pl.pallas_call(
        _gate_kernel,
        in_specs=[
            pl.BlockSpec(memory_space=pltpu.MemorySpace.HBM),
            pl.BlockSpec((_EXPERTS, _DIM), lambda: (0, 0)),
        ],
        out_specs=pl.BlockSpec(memory_space=pltpu.MemorySpace.HBM),
        out_shape=jax.ShapeDtypeStruct((_TOKENS, _EXPERTS), jnp.float32),
        scratch_shapes=[
            pltpu.VMEM((_NBUF, _BT, _DIM), jnp.float32),
            pltpu.VMEM((_NBUF, _BT, _EXPERTS), jnp.float32),
            pltpu.SemaphoreType.DMA((_NBUF,)),
            pltpu.SemaphoreType.DMA((_NBUF,)),
        ],
    )(inputs, wg)


# speedup vs baseline: 1.2455x; 1.0979x over previous
import jax
import jax.numpy as jnp
from jax.experimental import pallas as pl
from jax.experimental.pallas import tpu as pltpu

_TOKENS = 8192
_DIM = 2048
_EXPERTS = 64
_BT = 1024


def _gate_kernel(x_ref, w_ref, out_ref):
    logits = jax.lax.dot_general(
        x_ref[...], w_ref[...],
        dimension_numbers=(((1,), (1,)), ((), ())),
        preferred_element_type=jnp.float32)
    m = jnp.max(logits, axis=1, keepdims=True)
    e = jnp.exp(logits - m)
    out_ref[...] = e / jnp.sum(e, axis=1, keepdims=True)


def kernel(inputs, wg):
    return pl.pallas_call(
        _gate_kernel,
        grid=(_TOKENS // _BT,),
        in_specs=[
            pl.BlockSpec((_BT, _DIM), lambda i: (i, 0)),
            pl.BlockSpec((_EXPERTS, _DIM), lambda i: (0, 0)),
        ],
        out_specs=pl.BlockSpec((_BT, _EXPERTS), lambda i: (i, 0)),
        out_shape=jax.ShapeDtypeStruct((_TOKENS, _EXPERTS), jnp.float32),
        compiler_params=pltpu.CompilerParams(
            dimension_semantics=("parallel",)),
    )(inputs, wg)
